# baseline (speedup 1.0000x reference)
import functools

import jax
import jax.numpy as jnp
from jax import lax
from jax.experimental import pallas as pl
from jax.experimental.pallas import tpu as pltpu
from jax.experimental.pallas import tpu_sc as plsc

N = 4097
BR = 520
GRID = (N + BR - 1) // BR

NC, NS, L = 2, 16, 16
NW = NC * NS
ROWS_PER_W = (N - 1) // NW
CH = 8
NCHUNK = ROWS_PER_W // CH


CHW = CH * N


def _zero_buf(buf):
    z16 = jnp.zeros((L,), jnp.float32)

    def col(j, _):
        buf[pl.ds(j * L, L)] = z16
        return 0

    lax.fori_loop(0, CHW // L, col, 0)
    buf[pl.ds(CHW - L, L)] = z16


def _sc_alow_body(cl_hbm, cu_hbm, alow_hbm, clv, cuv, dlv, buf0, buf1):
    wid = lax.axis_index("s") * NC + lax.axis_index("c")
    base = wid * ROWS_PER_W

    _zero_buf(buf0)
    _zero_buf(buf1)

    pltpu.sync_copy(cl_hbm.at[pl.ds(base, ROWS_PER_W)], clv)
    pltpu.sync_copy(cu_hbm.at[pl.ds(base, ROWS_PER_W)], cuv)

    def dcol(j, _):
        cl = clv[pl.ds(j * L, L)]
        cu = cuv[pl.ds(j * L, L)]
        dead = cu <= 0.0
        pos = jnp.logical_and(~dead, cl >= 0.0)
        cross = jnp.logical_and(~dead, cl < 0.0)
        alpha = jnp.where(cu < -cl, jnp.float32(1e-5), jnp.float32(1.0))
        dlv[pl.ds(j * L, L)] = jnp.where(
            pos, jnp.float32(1.0), jnp.where(cross, alpha, jnp.float32(0.0))
        )
        return 0

    lax.fori_loop(0, ROWS_PER_W // L, dcol, 0)

    lane = lax.broadcasted_iota(jnp.int32, (L,), 0)
    lane8 = jnp.bitwise_and(lane, 7)
    in8 = lane < 8
    z16 = jnp.zeros((L,), jnp.float32)

    def pair(p, _):
        for b, buf in ((0, buf0), (1, buf1)):
            k = p * 2 + b
            r0 = base + k * CH
            vals = dlv[pl.ds(k * CH, L)]
            flat = lane8 * (N + 1) + r0
            plsc.store_scatter(buf, [flat], vals, mask=in8)
            pltpu.sync_copy(buf, alow_hbm.at[pl.ds(r0 * N, CHW)])
            plsc.store_scatter(buf, [flat], z16, mask=in8)
        return 0

    lax.fori_loop(0, NCHUNK // 2, pair, 0)

    @pl.when(wid == 0)
    def _():
        one_hot = jnp.where(lane == 0, jnp.float32(1.0), jnp.float32(0.0))
        buf0[pl.ds(N - 1, L)] = one_hot
        pltpu.sync_copy(buf0.at[pl.ds(0, N)], alow_hbm.at[pl.ds((N - 1) * N, N)])


def _tc_body(clp_ref, cup_ref, aup_ref, ocl_ref, ocu_ref):
    i = pl.program_id(0)
    cl = clp_ref[...]
    cu = cup_ref[...]

    dead = cu <= 0.0
    pos = jnp.logical_and(~dead, cl >= 0.0)
    cross = jnp.logical_and(~dead, cl < 0.0)

    alpha = jnp.where(cu < -cl, jnp.float32(1e-5), jnp.float32(1.0))
    denom = jnp.where(cross, cu - cl, jnp.float32(1.0))
    lam = jnp.where(cross, cu / denom, jnp.float32(0.0))

    zero = jnp.float32(0.0)
    one = jnp.float32(1.0)
    diag_up = jnp.where(pos, one, jnp.where(cross, lam, zero))
    bias_up = jnp.where(cross, -lam * cl, zero)

    rows = i * BR + jax.lax.broadcasted_iota(jnp.int32, (BR, N), 0)
    cols = jax.lax.broadcasted_iota(jnp.int32, (BR, N), 1)
    on_diag = rows == cols

    @pl.when(i == GRID - 1)
    def _():
        aup_ref[...] = jnp.where(
            on_diag, diag_up, jnp.where(rows == N - 1, bias_up, zero)
        )

    @pl.when(i != GRID - 1)
    def _():
        aup_ref[...] = jnp.where(on_diag, diag_up, zero)

    @pl.when(i == 0)
    def _():
        out_cl = jnp.where(pos, cl, jnp.where(cross, alpha * cl, zero))
        out_cu = jnp.where(dead, zero, cu)
        ocl_ref[...] = out_cl[:, : N - 1]
        ocu_ref[...] = out_cu[:, : N - 1]


@functools.partial(
    pl.kernel,
    mesh=plsc.VectorSubcoreMesh(core_axis_name="c", subcore_axis_name="s"),
    out_type=jax.ShapeDtypeStruct((N * N,), jnp.float32),
    scratch_types=[
        pltpu.VMEM((ROWS_PER_W,), jnp.float32),
        pltpu.VMEM((ROWS_PER_W,), jnp.float32),
        pltpu.VMEM((ROWS_PER_W + L,), jnp.float32),
        pltpu.VMEM((CHW,), jnp.float32),
        pltpu.VMEM((CHW,), jnp.float32),
    ],
    compiler_params=pltpu.CompilerParams(needs_layout_passes=False),
)
def _sc_alow(cl_hbm, cu_hbm, alow_hbm, clv, cuv, dlv, buf0, buf1):
    _sc_alow_body(cl_hbm, cu_hbm, alow_hbm, clv, cuv, dlv, buf0, buf1)


def kernel(concrete_lower, concrete_upper, abstract_lower_in, abstract_upper_in):
    n = N - 1
    a_low = _sc_alow(concrete_lower, concrete_upper).reshape(N, N)

    pad = jnp.ones((1, 1), dtype=jnp.float32)
    clp = jnp.concatenate([concrete_lower.reshape(1, n), pad], axis=1)
    cup = jnp.concatenate([concrete_upper.reshape(1, n), pad], axis=1)

    a_up, out_cl, out_cu = pl.pallas_call(
        _tc_body,
        grid=(GRID,),
        in_specs=[
            pl.BlockSpec((1, N), lambda i: (0, 0)),
            pl.BlockSpec((1, N), lambda i: (0, 0)),
        ],
        out_specs=[
            pl.BlockSpec((BR, N), lambda i: (i, 0)),
            pl.BlockSpec((1, n), lambda i: (0, 0)),
            pl.BlockSpec((1, n), lambda i: (0, 0)),
        ],
        out_shape=[
            jax.ShapeDtypeStruct((N, N), jnp.float32),
            jax.ShapeDtypeStruct((1, n), jnp.float32),
            jax.ShapeDtypeStruct((1, n), jnp.float32),
        ],
    )(clp, cup)
    return (out_cl.reshape(n), out_cu.reshape(n), a_low, a_up)


# speedup vs baseline: 3.3518x; 3.3518x over previous
import functools

import jax
import jax.numpy as jnp
from jax import lax
from jax.experimental import pallas as pl
from jax.experimental.pallas import tpu as pltpu
from jax.experimental.pallas import tpu_sc as plsc

N = 4097
BR = 520
GRID = (N + BR - 1) // BR

NC, NS, L = 2, 16, 16
NW = NC * NS
ROWS_PER_W = (N - 1) // NW
CH = 8
NCHUNK = ROWS_PER_W // CH


def _zero_buf(buf):
    z16 = jnp.zeros((L,), jnp.float32)

    for r in range(CH):
        def col(j, _, r=r):
            buf[r, pl.ds(j * L, L)] = z16
            return 0

        lax.fori_loop(0, (N - 1) // L, col, 0)
        buf[r, pl.ds(N - L, L)] = z16


def _sc_alow_body(cl_hbm, cu_hbm, alow_hbm, clv, cuv, dlv, buf0, buf1):
    wid = lax.axis_index("s") * NC + lax.axis_index("c")
    base = wid * ROWS_PER_W

    _zero_buf(buf0)
    _zero_buf(buf1)

    pltpu.sync_copy(cl_hbm.at[pl.ds(base, ROWS_PER_W)], clv)
    pltpu.sync_copy(cu_hbm.at[pl.ds(base, ROWS_PER_W)], cuv)

    def dcol(j, _):
        cl = clv[pl.ds(j * L, L)]
        cu = cuv[pl.ds(j * L, L)]
        dead = cu <= 0.0
        pos = jnp.logical_and(~dead, cl >= 0.0)
        cross = jnp.logical_and(~dead, cl < 0.0)
        alpha = jnp.where(cu < -cl, jnp.float32(1e-5), jnp.float32(1.0))
        dlv[pl.ds(j * L, L)] = jnp.where(
            pos, jnp.float32(1.0), jnp.where(cross, alpha, jnp.float32(0.0))
        )
        return 0

    lax.fori_loop(0, ROWS_PER_W // L, dcol, 0)

    lane = lax.broadcasted_iota(jnp.int32, (L,), 0)
    lane8 = jnp.bitwise_and(lane, 7)
    in8 = lane < 8
    z16 = jnp.zeros((L,), jnp.float32)

    def pair(p, _):
        for b, buf in ((0, buf0), (1, buf1)):
            k = p * 2 + b
            r0 = base + k * CH
            vals = dlv[pl.ds(k * CH, L)]
            cols = r0 + lane8
            plsc.store_scatter(buf, [lane8, cols], vals, mask=in8)
            pltpu.sync_copy(buf, alow_hbm.at[pl.ds(r0, CH)])
            plsc.store_scatter(buf, [lane8, cols], z16, mask=in8)
        return 0

    lax.fori_loop(0, NCHUNK // 2, pair, 0)

    @pl.when(wid == 0)
    def _():
        one_hot = jnp.where(lane == L - 1, jnp.float32(1.0), jnp.float32(0.0))
        buf0[0, pl.ds(N - L, L)] = one_hot
        pltpu.sync_copy(buf0.at[pl.ds(0, 1)], alow_hbm.at[pl.ds(N - 1, 1)])


def _tc_body(clp_ref, cup_ref, aup_ref, ocl_ref, ocu_ref):
    i = pl.program_id(0)
    cl = clp_ref[...]
    cu = cup_ref[...]

    dead = cu <= 0.0
    pos = jnp.logical_and(~dead, cl >= 0.0)
    cross = jnp.logical_and(~dead, cl < 0.0)

    alpha = jnp.where(cu < -cl, jnp.float32(1e-5), jnp.float32(1.0))
    denom = jnp.where(cross, cu - cl, jnp.float32(1.0))
    lam = jnp.where(cross, cu / denom, jnp.float32(0.0))

    zero = jnp.float32(0.0)
    one = jnp.float32(1.0)
    diag_up = jnp.where(pos, one, jnp.where(cross, lam, zero))
    bias_up = jnp.where(cross, -lam * cl, zero)

    rows = i * BR + jax.lax.broadcasted_iota(jnp.int32, (BR, N), 0)
    cols = jax.lax.broadcasted_iota(jnp.int32, (BR, N), 1)
    on_diag = rows == cols

    @pl.when(i == GRID - 1)
    def _():
        aup_ref[...] = jnp.where(
            on_diag, diag_up, jnp.where(rows == N - 1, bias_up, zero)
        )

    @pl.when(i != GRID - 1)
    def _():
        aup_ref[...] = jnp.where(on_diag, diag_up, zero)

    @pl.when(i == 0)
    def _():
        out_cl = jnp.where(pos, cl, jnp.where(cross, alpha * cl, zero))
        out_cu = jnp.where(dead, zero, cu)
        ocl_ref[...] = out_cl[:, : N - 1]
        ocu_ref[...] = out_cu[:, : N - 1]


@functools.partial(
    pl.kernel,
    mesh=plsc.VectorSubcoreMesh(core_axis_name="c", subcore_axis_name="s"),
    out_type=jax.ShapeDtypeStruct((N, N), jnp.float32),
    scratch_types=[
        pltpu.VMEM((ROWS_PER_W,), jnp.float32),
        pltpu.VMEM((ROWS_PER_W,), jnp.float32),
        pltpu.VMEM((ROWS_PER_W + L,), jnp.float32),
        pltpu.VMEM((CH, N), jnp.float32),
        pltpu.VMEM((CH, N), jnp.float32),
    ],
    compiler_params=pltpu.CompilerParams(
        needs_layout_passes=False, use_tc_tiling_on_sc=False
    ),
)
def _sc_alow(cl_hbm, cu_hbm, alow_hbm, clv, cuv, dlv, buf0, buf1):
    _sc_alow_body(cl_hbm, cu_hbm, alow_hbm, clv, cuv, dlv, buf0, buf1)


def kernel(concrete_lower, concrete_upper, abstract_lower_in, abstract_upper_in):
    n = N - 1
    a_low = _sc_alow(concrete_lower, concrete_upper)

    pad = jnp.ones((1, 1), dtype=jnp.float32)
    clp = jnp.concatenate([concrete_lower.reshape(1, n), pad], axis=1)
    cup = jnp.concatenate([concrete_upper.reshape(1, n), pad], axis=1)

    a_up, out_cl, out_cu = pl.pallas_call(
        _tc_body,
        grid=(GRID,),
        in_specs=[
            pl.BlockSpec((1, N), lambda i: (0, 0)),
            pl.BlockSpec((1, N), lambda i: (0, 0)),
        ],
        out_specs=[
            pl.BlockSpec((BR, N), lambda i: (i, 0)),
            pl.BlockSpec((1, n), lambda i: (0, 0)),
            pl.BlockSpec((1, n), lambda i: (0, 0)),
        ],
        out_shape=[
            jax.ShapeDtypeStruct((N, N), jnp.float32),
            jax.ShapeDtypeStruct((1, n), jnp.float32),
            jax.ShapeDtypeStruct((1, n), jnp.float32),
        ],
    )(clp, cup)
    return (out_cl.reshape(n), out_cu.reshape(n), a_low, a_up)


# speedup vs baseline: 9.0751x; 2.7075x over previous
import jax
import jax.numpy as jnp
from jax.experimental import pallas as pl

N = 4097
BR = 256
GRID = (N + BR - 1) // BR


def _relu_body(clp_ref, cup_ref, alow_ref, aup_ref, ocl_ref, ocu_ref):
    i = pl.program_id(0)
    cl = clp_ref[...]
    cu = cup_ref[...]

    dead = cu <= 0.0
    pos = jnp.logical_and(~dead, cl >= 0.0)
    cross = jnp.logical_and(~dead, cl < 0.0)

    alpha = jnp.where(cu < -cl, jnp.float32(1e-5), jnp.float32(1.0))
    denom = jnp.where(cross, cu - cl, jnp.float32(1.0))
    lam = jnp.where(cross, cu / denom, jnp.float32(0.0))

    zero = jnp.float32(0.0)
    one = jnp.float32(1.0)
    diag_low = jnp.where(pos, one, jnp.where(cross, alpha, zero))
    diag_up = jnp.where(pos, one, jnp.where(cross, lam, zero))
    bias_up = jnp.where(cross, -lam * cl, zero)

    rows = i * BR + jax.lax.broadcasted_iota(jnp.int32, (BR, N), 0)
    cols = jax.lax.broadcasted_iota(jnp.int32, (BR, N), 1)
    on_diag = rows == cols

    alow_ref[...] = jnp.where(on_diag, diag_low, zero)

    @pl.when(i == GRID - 1)
    def _():
        aup_ref[...] = jnp.where(
            on_diag, diag_up, jnp.where(rows == N - 1, bias_up, zero)
        )

    @pl.when(i != GRID - 1)
    def _():
        aup_ref[...] = jnp.where(on_diag, diag_up, zero)

    @pl.when(i == 0)
    def _():
        out_cl = jnp.where(pos, cl, jnp.where(cross, alpha * cl, zero))
        out_cu = jnp.where(dead, zero, cu)
        ocl_ref[...] = out_cl[:, : N - 1]
        ocu_ref[...] = out_cu[:, : N - 1]


def kernel(concrete_lower, concrete_upper, abstract_lower_in, abstract_upper_in):
    n = N - 1
    pad = jnp.ones((1, 1), dtype=jnp.float32)
    clp = jnp.concatenate([concrete_lower.reshape(1, n), pad], axis=1)
    cup = jnp.concatenate([concrete_upper.reshape(1, n), pad], axis=1)

    a_low, a_up, out_cl, out_cu = pl.pallas_call(
        _relu_body,
        grid=(GRID,),
        in_specs=[
            pl.BlockSpec((1, N), lambda i: (0, 0)),
            pl.BlockSpec((1, N), lambda i: (0, 0)),
        ],
        out_specs=[
            pl.BlockSpec((BR, N), lambda i: (i, 0)),
            pl.BlockSpec((BR, N), lambda i: (i, 0)),
            pl.BlockSpec((1, n), lambda i: (0, 0)),
            pl.BlockSpec((1, n), lambda i: (0, 0)),
        ],
        out_shape=[
            jax.ShapeDtypeStruct((N, N), jnp.float32),
            jax.ShapeDtypeStruct((N, N), jnp.float32),
            jax.ShapeDtypeStruct((1, n), jnp.float32),
            jax.ShapeDtypeStruct((1, n), jnp.float32),
        ],
    )(clp, cup)
    return (out_cl.reshape(n), out_cu.reshape(n), a_low, a_up)


# speedup vs baseline: 9.2664x; 1.0211x over previous
import jax
import jax.numpy as jnp
from jax.experimental import pallas as pl

N = 4097
BR = 128
GRID = (N + BR - 1) // BR


def _relu_body(clp_ref, cup_ref, alow_ref, aup_ref, ocl_ref, ocu_ref):
    i = pl.program_id(0)
    cl = clp_ref[...]
    cu = cup_ref[...]

    dead = cu <= 0.0
    pos = jnp.logical_and(~dead, cl >= 0.0)
    cross = jnp.logical_and(~dead, cl < 0.0)

    alpha = jnp.where(cu < -cl, jnp.float32(1e-5), jnp.float32(1.0))
    denom = jnp.where(cross, cu - cl, jnp.float32(1.0))
    lam = jnp.where(cross, cu / denom, jnp.float32(0.0))

    zero = jnp.float32(0.0)
    one = jnp.float32(1.0)
    diag_low = jnp.where(pos, one, jnp.where(cross, alpha, zero))
    diag_up = jnp.where(pos, one, jnp.where(cross, lam, zero))
    bias_up = jnp.where(cross, -lam * cl, zero)

    rows = i * BR + jax.lax.broadcasted_iota(jnp.int32, (BR, N), 0)
    cols = jax.lax.broadcasted_iota(jnp.int32, (BR, N), 1)
    on_diag = rows == cols

    alow_ref[...] = jnp.where(on_diag, diag_low, zero)

    @pl.when(i == GRID - 1)
    def _():
        aup_ref[...] = jnp.where(
            on_diag, diag_up, jnp.where(rows == N - 1, bias_up, zero)
        )

    @pl.when(i != GRID - 1)
    def _():
        aup_ref[...] = jnp.where(on_diag, diag_up, zero)

    @pl.when(i == 0)
    def _():
        out_cl = jnp.where(pos, cl, jnp.where(cross, alpha * cl, zero))
        out_cu = jnp.where(dead, zero, cu)
        ocl_ref[...] = out_cl[:, : N - 1]
        ocu_ref[...] = out_cu[:, : N - 1]


def kernel(concrete_lower, concrete_upper, abstract_lower_in, abstract_upper_in):
    n = N - 1
    pad = jnp.ones((1, 1), dtype=jnp.float32)
    clp = jnp.concatenate([concrete_lower.reshape(1, n), pad], axis=1)
    cup = jnp.concatenate([concrete_upper.reshape(1, n), pad], axis=1)

    a_low, a_up, out_cl, out_cu = pl.pallas_call(
        _relu_body,
        grid=(GRID,),
        in_specs=[
            pl.BlockSpec((1, N), lambda i: (0, 0)),
            pl.BlockSpec((1, N), lambda i: (0, 0)),
        ],
        out_specs=[
            pl.BlockSpec((BR, N), lambda i: (i, 0)),
            pl.BlockSpec((BR, N), lambda i: (i, 0)),
            pl.BlockSpec((1, n), lambda i: (0, 0)),
            pl.BlockSpec((1, n), lambda i: (0, 0)),
        ],
        out_shape=[
            jax.ShapeDtypeStruct((N, N), jnp.float32),
            jax.ShapeDtypeStruct((N, N), jnp.float32),
            jax.ShapeDtypeStruct((1, n), jnp.float32),
            jax.ShapeDtypeStruct((1, n), jnp.float32),
        ],
    )(clp, cup)
    return (out_cl.reshape(n), out_cu.reshape(n), a_low, a_up)
